# baseline (speedup 1.0000x reference)
import functools

import jax
import jax.numpy as jnp
from jax import lax
from jax.experimental import pallas as pl
from jax.experimental.pallas import tpu as pltpu
from jax.experimental.pallas import tpu_sc as plsc

NC = 2
NS = 16
NW = NC * NS
LANES = 16
CH = 128
G_IDX = 9


def _deg_kernel(E, N):
    e_per_w = E // NW
    mesh = plsc.VectorSubcoreMesh(core_axis_name="c", subcore_axis_name="s")

    @functools.partial(
        pl.kernel,
        out_type=jax.ShapeDtypeStruct((NW, 1, N), jnp.float32),
        mesh=mesh,
        scratch_types=[
            pltpu.VMEM((e_per_w,), jnp.int32),
            pltpu.VMEM((N,), jnp.float32),
        ],
        compiler_params=pltpu.CompilerParams(needs_layout_passes=False),
    )
    def deg_kernel(dst_hbm, hist_hbm, dst_v, hist_v):
        c = lax.axis_index("c")
        s = lax.axis_index("s")
        wid = s * NC + c

        zero16 = jnp.zeros((LANES,), jnp.float32)

        def zbody(i, carry):
            hist_v[pl.ds(i * LANES, LANES)] = zero16
            return carry

        lax.fori_loop(0, N // LANES, zbody, 0)

        pltpu.sync_copy(dst_hbm.at[pl.ds(wid * e_per_w, e_per_w)], dst_v)

        ones16 = jnp.ones((LANES,), jnp.float32)

        def body(i, carry):
            idx = dst_v[pl.ds(i * LANES, LANES)]
            plsc.addupdate_scatter(hist_v, [idx], ones16)
            return carry

        lax.fori_loop(0, e_per_w // LANES, body, 0)

        pltpu.sync_copy(hist_v, hist_hbm.at[wid, 0])

    return deg_kernel


def _agg_kernel(N, D, E):
    e_per_w = E // NW
    n_chunks = e_per_w // CH
    rows_per_tile = ((N + NS - 1) // NS + 7) // 8 * 8
    n_pad = rows_per_tile * NS
    assert n_pad >= N + 1
    mesh = plsc.VectorSubcoreMesh(core_axis_name="c", subcore_axis_name="s")

    G = G_IDX
    gch = n_chunks // G
    assert n_chunks % G == 0 and gch % 2 == 1

    @functools.partial(
        pl.kernel,
        out_type=jax.ShapeDtypeStruct((NC, n_pad, D), jnp.float32),
        mesh=mesh,
        scratch_types=[
            pltpu.VMEM((2, gch, 1, CH), jnp.int32),
            pltpu.VMEM((2, gch, 1, CH), jnp.int32),
            pltpu.VMEM((CH, D), jnp.float32),
            pltpu.VMEM((CH, D), jnp.float32),
            pltpu.VMEM_SHARED((n_pad, D), jnp.float32),
            pltpu.SemaphoreType.DMA,
            pltpu.SemaphoreType.DMA,
            pltpu.SemaphoreType.DMA,
        ],
        compiler_params=pltpu.CompilerParams(needs_layout_passes=False),
    )
    def agg_kernel(src_hbm, dst_hbm, y_hbm, out_hbm, sidx_v, didx_v, buf_a,
                   buf_b, acc_sh, sem_a, sem_b, sem_i):
        c = lax.axis_index("c")
        s = lax.axis_index("s")
        wid = s * NC + c
        row0 = s * rows_per_tile

        def idxfetch(g, b):
            sl = pl.ds(g * gch, gch)
            pltpu.async_copy(src_hbm.at[wid, sl], sidx_v.at[b], sem_i)
            pltpu.async_copy(dst_hbm.at[wid, sl], didx_v.at[b], sem_i)

        def idxwait(b):
            sl = pl.ds(0, gch)
            pltpu.make_async_copy(src_hbm.at[wid, sl], sidx_v.at[b], sem_i).wait()
            pltpu.make_async_copy(dst_hbm.at[wid, sl], didx_v.at[b], sem_i).wait()

        def gather(b, i, buf, sem):
            pltpu.async_copy(y_hbm.at[sidx_v.at[b, i, 0]], buf, sem)

        def drain(buf, sem):
            pltpu.make_async_copy(y_hbm.at[pl.ds(0, CH)], buf, sem).wait()

        def scatter(b, i, buf):
            pltpu.sync_copy(buf, acc_sh.at[didx_v.at[b, i, 0]], add=True)

        idxfetch(0, 0)

        zero16 = jnp.zeros((LANES,), jnp.float32)

        def zrow(r, carry):
            for k in range(D // LANES):
                buf_a[r, pl.ds(k * LANES, LANES)] = zero16
            return carry

        lax.fori_loop(0, CH, zrow, 0)

        full = rows_per_tile // CH
        rem = rows_per_tile % CH
        for j in range(full):
            pltpu.sync_copy(buf_a, acc_sh.at[pl.ds(row0 + j * CH, CH)])
        if rem:
            pltpu.sync_copy(
                buf_a.at[pl.ds(0, rem)],
                acc_sh.at[pl.ds(row0 + full * CH, rem)],
            )
        idxwait(0)
        plsc.subcore_barrier()

        gather(0, 0, buf_a, sem_a)
        for g in range(G):
            b = g % 2
            if g + 1 < G:
                idxfetch(g + 1, 1 - b)

            def pair(j, carry, b=b):
                i0 = 2 * j
                gather(b, i0 + 1, buf_b, sem_b)
                drain(buf_a, sem_a)
                scatter(b, i0, buf_a)
                gather(b, i0 + 2, buf_a, sem_a)
                drain(buf_b, sem_b)
                scatter(b, i0 + 1, buf_b)
                return carry

            lax.fori_loop(0, (gch - 1) // 2, pair, 0)
            drain(buf_a, sem_a)
            scatter(b, gch - 1, buf_a)
            if g + 1 < G:
                idxwait(1 - b)
                gather(1 - b, 0, buf_a, sem_a)
        plsc.subcore_barrier()

        pltpu.sync_copy(
            acc_sh.at[pl.ds(row0, rows_per_tile)],
            out_hbm.at[c, pl.ds(row0, rows_per_tile)],
        )

    return agg_kernel


def _tc_transform(x, W, histT):
    N, _ = x.shape
    Dout = W.shape[1]

    def body(x_ref, w_ref, h_ref, y_ref, dis_ref):
        deg = jnp.sum(h_ref[...], axis=1, keepdims=True) + 1.0
        dis = lax.rsqrt(deg)
        xw = jnp.dot(x_ref[...], w_ref[...], preferred_element_type=jnp.float32)
        y_ref[...] = xw * dis
        dis_ref[...] = dis

    return pl.pallas_call(
        body,
        out_shape=(
            jax.ShapeDtypeStruct((N, Dout), jnp.float32),
            jax.ShapeDtypeStruct((N, 1), jnp.float32),
        ),
    )(x, W, histT)


def _tc_combine(agg, y, dis):
    N, D = y.shape

    def body(a_ref, y_ref, d_ref, o_ref):
        a = (a_ref[0] + a_ref[1])[:N]
        o_ref[...] = d_ref[...] * (a + y_ref[...])

    return pl.pallas_call(
        body,
        out_shape=jax.ShapeDtypeStruct((N, D), jnp.float32),
    )(agg, y, dis)


def kernel(x, edge_index, W):
    N, _ = x.shape
    Dout = W.shape[1]
    E = edge_index.shape[1]
    assert E % NW == 0 and (E // NW) % LANES == 0
    assert N % NS == 0 and N % LANES == 0 and Dout % LANES == 0

    ei = edge_index.astype(jnp.int32)
    srcs = ei[0]
    dsts = ei[1]

    hist = _deg_kernel(E, N)(dsts)
    histT = hist.reshape(NW, N).T
    y, dis = _tc_transform(x, W, histT)

    n_chunks = -(-E // (NW * CH))
    n_chunks = -(-n_chunks // G_IDX) * G_IDX
    e_pad = NW * n_chunks * CH
    pad = e_pad - E
    srcs_p = jnp.concatenate([srcs, jnp.zeros((pad,), jnp.int32)])
    dsts_p = jnp.concatenate([dsts, jnp.full((pad,), N, jnp.int32)])
    src3 = srcs_p.reshape(NW, n_chunks, 1, CH)
    dst3 = dsts_p.reshape(NW, n_chunks, 1, CH)
    agg = _agg_kernel(N, Dout, e_pad)(src3, dst3, y)
    return _tc_combine(agg, y, dis)


# speedup vs baseline: 2.4554x; 2.4554x over previous
import functools

import jax
import jax.numpy as jnp
from jax import lax
from jax.experimental import pallas as pl
from jax.experimental.pallas import tpu as pltpu
from jax.experimental.pallas import tpu_sc as plsc

NC = 2
NS = 16
NW = NC * NS
LANES = 16
CH = 120
G_IDX = 4


def _deg_kernel(E, N):
    e_per_w = E // NW
    mesh = plsc.VectorSubcoreMesh(core_axis_name="c", subcore_axis_name="s")

    @functools.partial(
        pl.kernel,
        out_type=jax.ShapeDtypeStruct((NW, 1, N), jnp.float32),
        mesh=mesh,
        scratch_types=[
            pltpu.VMEM((e_per_w,), jnp.int32),
            pltpu.VMEM((N,), jnp.float32),
        ],
        compiler_params=pltpu.CompilerParams(needs_layout_passes=False),
    )
    def deg_kernel(dst_hbm, hist_hbm, dst_v, hist_v):
        c = lax.axis_index("c")
        s = lax.axis_index("s")
        wid = s * NC + c

        zero16 = jnp.zeros((LANES,), jnp.float32)

        def zbody(i, carry):
            hist_v[pl.ds(i * LANES, LANES)] = zero16
            return carry

        lax.fori_loop(0, N // LANES, zbody, 0)

        pltpu.sync_copy(dst_hbm.at[pl.ds(wid * e_per_w, e_per_w)], dst_v)

        ones16 = jnp.ones((LANES,), jnp.float32)

        def body(i, carry):
            idx = dst_v[pl.ds(i * LANES, LANES)]
            plsc.addupdate_scatter(hist_v, [idx], ones16)
            return carry

        lax.fori_loop(0, e_per_w // LANES, body, 0)

        pltpu.sync_copy(hist_v, hist_hbm.at[wid, 0])

    return deg_kernel


def _agg_kernel(N, D, E):
    e_per_w = E // NW
    n_chunks = e_per_w // CH
    rows_per_tile = ((N + NS - 1) // NS + 7) // 8 * 8
    n_pad = rows_per_tile * NS
    assert n_pad >= N + 1
    mesh = plsc.VectorSubcoreMesh(core_axis_name="c", subcore_axis_name="s")

    G = G_IDX
    gch = n_chunks // G
    assert n_chunks % G == 0 and gch % 2 == 1

    @functools.partial(
        pl.kernel,
        out_type=jax.ShapeDtypeStruct((NC, n_pad, D), jnp.float32),
        mesh=mesh,
        scratch_types=[
            pltpu.VMEM((2, gch, 1, CH), jnp.int32),
            pltpu.VMEM((2, gch, 1, CH), jnp.int32),
            pltpu.VMEM((CH, D), jnp.float32),
            pltpu.VMEM((CH, D), jnp.float32),
            pltpu.VMEM_SHARED((n_pad, D), jnp.float32),
            pltpu.SemaphoreType.DMA,
            pltpu.SemaphoreType.DMA,
            pltpu.SemaphoreType.DMA,
        ],
        compiler_params=pltpu.CompilerParams(needs_layout_passes=False),
    )
    def agg_kernel(src_hbm, dst_hbm, y_hbm, out_hbm, sidx_v, didx_v, buf_a,
                   buf_b, acc_sh, sem_a, sem_b, sem_i):
        c = lax.axis_index("c")
        s = lax.axis_index("s")
        wid = s * NC + c
        row0 = s * rows_per_tile

        def idxfetch(g, b):
            sl = pl.ds(g * gch, gch)
            pltpu.async_copy(src_hbm.at[wid, sl], sidx_v.at[b], sem_i)
            pltpu.async_copy(dst_hbm.at[wid, sl], didx_v.at[b], sem_i)

        def idxwait(b):
            sl = pl.ds(0, gch)
            pltpu.make_async_copy(src_hbm.at[wid, sl], sidx_v.at[b], sem_i).wait()
            pltpu.make_async_copy(dst_hbm.at[wid, sl], didx_v.at[b], sem_i).wait()

        def gather(b, i, buf, sem):
            pltpu.async_copy(y_hbm.at[sidx_v.at[b, i, 0]], buf, sem)

        def drain(buf, sem):
            pltpu.make_async_copy(y_hbm.at[pl.ds(0, CH)], buf, sem).wait()

        def scatter(b, i, buf):
            pltpu.sync_copy(buf, acc_sh.at[didx_v.at[b, i, 0]], add=True)

        idxfetch(0, 0)

        zero16 = jnp.zeros((LANES,), jnp.float32)

        def zrow(r, carry):
            for k in range(D // LANES):
                buf_a[r, pl.ds(k * LANES, LANES)] = zero16
            return carry

        lax.fori_loop(0, CH, zrow, 0)

        full = rows_per_tile // CH
        rem = rows_per_tile % CH
        for j in range(full):
            pltpu.sync_copy(buf_a, acc_sh.at[pl.ds(row0 + j * CH, CH)])
        if rem:
            pltpu.sync_copy(
                buf_a.at[pl.ds(0, rem)],
                acc_sh.at[pl.ds(row0 + full * CH, rem)],
            )
        idxwait(0)
        plsc.subcore_barrier()

        gather(0, 0, buf_a, sem_a)
        for g in range(G):
            b = g % 2
            if g + 1 < G:
                idxfetch(g + 1, 1 - b)

            def pair(j, carry, b=b):
                i0 = 2 * j
                gather(b, i0 + 1, buf_b, sem_b)
                drain(buf_a, sem_a)
                scatter(b, i0, buf_a)
                gather(b, i0 + 2, buf_a, sem_a)
                drain(buf_b, sem_b)
                scatter(b, i0 + 1, buf_b)
                return carry

            lax.fori_loop(0, (gch - 1) // 2, pair, 0)
            drain(buf_a, sem_a)
            scatter(b, gch - 1, buf_a)
            if g + 1 < G:
                idxwait(1 - b)
                gather(1 - b, 0, buf_a, sem_a)
        plsc.subcore_barrier()

        pltpu.sync_copy(
            acc_sh.at[pl.ds(row0, rows_per_tile)],
            out_hbm.at[c, pl.ds(row0, rows_per_tile)],
        )

    return agg_kernel


def _tc_transform(x, W, histT):
    N, _ = x.shape
    Dout = W.shape[1]

    def body(x_ref, w_ref, h_ref, y_ref, dis_ref):
        deg = jnp.sum(h_ref[...], axis=1, keepdims=True) + 1.0
        dis = lax.rsqrt(deg)
        xw = jnp.dot(x_ref[...], w_ref[...], preferred_element_type=jnp.float32)
        y_ref[...] = xw * dis
        dis_ref[...] = dis

    return pl.pallas_call(
        body,
        out_shape=(
            jax.ShapeDtypeStruct((N, Dout), jnp.float32),
            jax.ShapeDtypeStruct((N, 1), jnp.float32),
        ),
    )(x, W, histT)


def _tc_combine(agg, y, dis):
    N, D = y.shape

    def body(a_ref, y_ref, d_ref, o_ref):
        a = (a_ref[0] + a_ref[1])[:N]
        o_ref[...] = d_ref[...] * (a + y_ref[...])

    return pl.pallas_call(
        body,
        out_shape=jax.ShapeDtypeStruct((N, D), jnp.float32),
    )(agg, y, dis)


def kernel(x, edge_index, W):
    N, _ = x.shape
    Dout = W.shape[1]
    E = edge_index.shape[1]
    assert E % NW == 0 and (E // NW) % LANES == 0
    assert N % NS == 0 and N % LANES == 0 and Dout % LANES == 0

    ei = edge_index.astype(jnp.int32)
    srcs = ei[0]
    dsts = ei[1]

    hist = _deg_kernel(E, N)(dsts)
    histT = hist.reshape(NW, N).T
    y, dis = _tc_transform(x, W, histT)

    n_chunks = -(-E // (NW * CH))
    n_chunks = -(-n_chunks // G_IDX) * G_IDX
    e_pad = NW * n_chunks * CH
    pad = e_pad - E
    srcs_p = jnp.concatenate([srcs, jnp.zeros((pad,), jnp.int32)])
    dsts_p = jnp.concatenate([dsts, jnp.full((pad,), N, jnp.int32)])
    src3 = srcs_p.reshape(NW, n_chunks, 1, CH)
    dst3 = dsts_p.reshape(NW, n_chunks, 1, CH)
    agg = _agg_kernel(N, Dout, e_pad)(src3, dst3, y)
    return _tc_combine(agg, y, dis)


# speedup vs baseline: 2.9463x; 1.1999x over previous
import functools

import jax
import jax.numpy as jnp
from jax import lax
from jax.experimental import pallas as pl
from jax.experimental.pallas import tpu as pltpu
from jax.experimental.pallas import tpu_sc as plsc

NC = 2
NS = 16
NW = NC * NS
LANES = 16
CH = 120
G_IDX = 4


def _deg_kernel(E, N):
    e_per_w = E // NW
    mesh = plsc.VectorSubcoreMesh(core_axis_name="c", subcore_axis_name="s")

    @functools.partial(
        pl.kernel,
        out_type=jax.ShapeDtypeStruct((NW, 1, N), jnp.float32),
        mesh=mesh,
        scratch_types=[
            pltpu.VMEM((e_per_w,), jnp.int32),
            pltpu.VMEM((N,), jnp.float32),
        ],
        compiler_params=pltpu.CompilerParams(needs_layout_passes=False),
    )
    def deg_kernel(dst_hbm, hist_hbm, dst_v, hist_v):
        c = lax.axis_index("c")
        s = lax.axis_index("s")
        wid = s * NC + c

        zero16 = jnp.zeros((LANES,), jnp.float32)

        def zbody(i, carry):
            hist_v[pl.ds(i * LANES, LANES)] = zero16
            return carry

        lax.fori_loop(0, N // LANES, zbody, 0)

        pltpu.sync_copy(dst_hbm.at[pl.ds(wid * e_per_w, e_per_w)], dst_v)

        ones16 = jnp.ones((LANES,), jnp.float32)

        def body(i, carry):
            idx = dst_v[pl.ds(i * LANES, LANES)]
            plsc.addupdate_scatter(hist_v, [idx], ones16)
            return carry

        lax.fori_loop(0, e_per_w // LANES, body, 0)

        pltpu.sync_copy(hist_v, hist_hbm.at[wid, 0])

    return deg_kernel


def _agg_kernel(N, D, E):
    e_per_w = E // NW
    n_chunks = e_per_w // CH
    rows_per_tile = ((N + NS - 1) // NS + 7) // 8 * 8
    n_pad = rows_per_tile * NS
    assert n_pad >= N + 1
    mesh = plsc.VectorSubcoreMesh(core_axis_name="c", subcore_axis_name="s")

    G = G_IDX
    gch = n_chunks // G
    assert n_chunks % G == 0 and gch % 2 == 1

    @functools.partial(
        pl.kernel,
        out_type=jax.ShapeDtypeStruct((NC, n_pad, D), jnp.float32),
        mesh=mesh,
        scratch_types=[
            pltpu.VMEM((2, gch, 1, CH), jnp.int32),
            pltpu.VMEM((2, gch, 1, CH), jnp.int32),
            pltpu.VMEM((CH, D), jnp.float32),
            pltpu.VMEM((CH, D), jnp.float32),
            pltpu.VMEM_SHARED((n_pad, D), jnp.float32),
            pltpu.SemaphoreType.DMA,
            pltpu.SemaphoreType.DMA,
            pltpu.SemaphoreType.DMA,
        ],
        compiler_params=pltpu.CompilerParams(needs_layout_passes=False),
    )
    def agg_kernel(src_hbm, dst_hbm, y_hbm, out_hbm, sidx_v, didx_v, buf_a,
                   buf_b, acc_sh, sem_a, sem_b, sem_i):
        c = lax.axis_index("c")
        s = lax.axis_index("s")
        wid = s * NC + c
        row0 = s * rows_per_tile

        def idxfetch(g, b):
            sl = pl.ds(g * gch, gch)
            pltpu.async_copy(src_hbm.at[wid, sl], sidx_v.at[b], sem_i)
            pltpu.async_copy(dst_hbm.at[wid, sl], didx_v.at[b], sem_i)

        def idxwait(b):
            sl = pl.ds(0, gch)
            pltpu.make_async_copy(src_hbm.at[wid, sl], sidx_v.at[b], sem_i).wait()
            pltpu.make_async_copy(dst_hbm.at[wid, sl], didx_v.at[b], sem_i).wait()

        def gather(b, i, buf, sem):
            pltpu.async_copy(y_hbm.at[sidx_v.at[b, i, 0]], buf, sem)

        def drain(buf, sem):
            pltpu.make_async_copy(y_hbm.at[pl.ds(0, CH)], buf, sem).wait()

        def scatter(b, i, buf):
            pltpu.sync_copy(buf, acc_sh.at[didx_v.at[b, i, 0]], add=True)

        idxfetch(0, 0)

        zero16 = jnp.zeros((LANES,), jnp.float32)

        def zrow(r, carry):
            for k in range(D // LANES):
                buf_a[r, pl.ds(k * LANES, LANES)] = zero16
            return carry

        lax.fori_loop(0, CH, zrow, 0)

        full = rows_per_tile // CH
        rem = rows_per_tile % CH
        for j in range(full):
            pltpu.sync_copy(buf_a, acc_sh.at[pl.ds(row0 + j * CH, CH)])
        if rem:
            pltpu.sync_copy(
                buf_a.at[pl.ds(0, rem)],
                acc_sh.at[pl.ds(row0 + full * CH, rem)],
            )
        idxwait(0)
        plsc.subcore_barrier()

        gather(0, 0, buf_a, sem_a)
        for g in range(G):
            b = g % 2
            if g + 1 < G:
                idxfetch(g + 1, 1 - b)

            def pair(j, carry, b=b):
                i0 = 2 * j
                gather(b, i0 + 1, buf_b, sem_b)
                drain(buf_a, sem_a)
                scatter(b, i0, buf_a)
                gather(b, i0 + 2, buf_a, sem_a)
                drain(buf_b, sem_b)
                scatter(b, i0 + 1, buf_b)
                return carry

            lax.fori_loop(0, (gch - 1) // 2, pair, 0)
            drain(buf_a, sem_a)
            scatter(b, gch - 1, buf_a)
            if g + 1 < G:
                idxwait(1 - b)
                gather(1 - b, 0, buf_a, sem_a)
        plsc.subcore_barrier()

        pltpu.sync_copy(
            acc_sh.at[pl.ds(row0, rows_per_tile)],
            out_hbm.at[c, pl.ds(row0, rows_per_tile)],
        )

    return agg_kernel


def _tc_transform(x, W, histT):
    N, _ = x.shape
    Dout = W.shape[1]

    def body(x_ref, w_ref, h_ref, y_ref, dis_ref):
        deg = jnp.sum(h_ref[...], axis=1, keepdims=True) + 1.0
        dis = lax.rsqrt(deg)
        xw = jnp.dot(x_ref[...], w_ref[...], preferred_element_type=jnp.float32)
        y_ref[...] = xw * dis
        dis_ref[...] = dis

    return pl.pallas_call(
        body,
        out_shape=(
            jax.ShapeDtypeStruct((N, Dout), jnp.float32),
            jax.ShapeDtypeStruct((N, 1), jnp.float32),
        ),
    )(x, W, histT)


def _tc_combine(agg, y, dis):
    N, D = y.shape

    def body(a_ref, y_ref, d_ref, o_ref):
        a = (a_ref[0] + a_ref[1])[:N]
        o_ref[...] = d_ref[...] * (a + y_ref[...])

    return pl.pallas_call(
        body,
        out_shape=jax.ShapeDtypeStruct((N, D), jnp.float32),
    )(agg, y, dis)


def kernel(x, edge_index, W):
    N, _ = x.shape
    Dout = W.shape[1]
    E = edge_index.shape[1]
    assert E % NW == 0 and (E // NW) % LANES == 0
    assert N % NS == 0 and N % LANES == 0 and Dout % LANES == 0

    ei = edge_index.astype(jnp.int32)
    srcs = ei[0]
    dsts = ei[1]

    hist = _deg_kernel(E, N)(dsts)
    histT = hist.reshape(NW, N).T
    y, dis = _tc_transform(x, W, histT)

    e_per_w = E // NW
    n_chunks = -(-e_per_w // CH)
    n_chunks = -(-n_chunks // G_IDX) * G_IDX
    ppw = n_chunks * CH - e_per_w
    rows_per_tile = ((N + NS - 1) // NS + 7) // 8 * 8
    n_spare = rows_per_tile * NS - N
    pad_src = jnp.zeros((NW, ppw), jnp.int32)
    pad_dst = jnp.broadcast_to(
        N + (jnp.arange(NW, dtype=jnp.int32) % n_spare)[:, None], (NW, ppw)
    )
    src3 = jnp.concatenate([srcs.reshape(NW, e_per_w), pad_src], axis=1)
    dst3 = jnp.concatenate([dsts.reshape(NW, e_per_w), pad_dst], axis=1)
    src3 = src3.reshape(NW, n_chunks, 1, CH)
    dst3 = dst3.reshape(NW, n_chunks, 1, CH)
    agg = _agg_kernel(N, Dout, NW * n_chunks * CH)(src3, dst3, y)
    return _tc_combine(agg, y, dis)


# speedup vs baseline: 3.2861x; 1.1153x over previous
import functools

import jax
import jax.numpy as jnp
from jax import lax
from jax.experimental import pallas as pl
from jax.experimental.pallas import tpu as pltpu
from jax.experimental.pallas import tpu_sc as plsc

NC = 2
NS = 16
NW = NC * NS
LANES = 16
CH = 40
G_IDX = 10


def _deg_kernel(E, N):
    e_per_w = E // NW
    mesh = plsc.VectorSubcoreMesh(core_axis_name="c", subcore_axis_name="s")

    @functools.partial(
        pl.kernel,
        out_type=jax.ShapeDtypeStruct((NW, 1, N), jnp.float32),
        mesh=mesh,
        scratch_types=[
            pltpu.VMEM((e_per_w,), jnp.int32),
            pltpu.VMEM((N,), jnp.float32),
        ],
        compiler_params=pltpu.CompilerParams(needs_layout_passes=False),
    )
    def deg_kernel(dst_hbm, hist_hbm, dst_v, hist_v):
        c = lax.axis_index("c")
        s = lax.axis_index("s")
        wid = s * NC + c

        zero16 = jnp.zeros((LANES,), jnp.float32)

        def zbody(i, carry):
            hist_v[pl.ds(i * LANES, LANES)] = zero16
            return carry

        lax.fori_loop(0, N // LANES, zbody, 0)

        pltpu.sync_copy(dst_hbm.at[pl.ds(wid * e_per_w, e_per_w)], dst_v)

        ones16 = jnp.ones((LANES,), jnp.float32)

        def body(i, carry):
            idx = dst_v[pl.ds(i * LANES, LANES)]
            plsc.addupdate_scatter(hist_v, [idx], ones16)
            return carry

        lax.fori_loop(0, e_per_w // LANES, body, 0)

        pltpu.sync_copy(hist_v, hist_hbm.at[wid, 0])

    return deg_kernel


def _agg_kernel(N, D, E):
    e_per_w = E // NW
    n_chunks = e_per_w // CH
    rows_per_tile = ((N + NS - 1) // NS + 7) // 8 * 8
    n_pad = rows_per_tile * NS
    assert n_pad >= N + 1
    mesh = plsc.VectorSubcoreMesh(core_axis_name="c", subcore_axis_name="s")

    G = G_IDX
    gch = n_chunks // G
    assert n_chunks % G == 0 and gch % 2 == 1

    @functools.partial(
        pl.kernel,
        out_type=jax.ShapeDtypeStruct((NC, n_pad, D), jnp.float32),
        mesh=mesh,
        scratch_types=[
            pltpu.VMEM((2, gch, 1, CH), jnp.int32),
            pltpu.VMEM((2, gch, 1, CH), jnp.int32),
            pltpu.VMEM((CH, D), jnp.float32),
            pltpu.VMEM((CH, D), jnp.float32),
            pltpu.VMEM_SHARED((n_pad, D), jnp.float32),
            pltpu.SemaphoreType.DMA,
            pltpu.SemaphoreType.DMA,
            pltpu.SemaphoreType.DMA,
        ],
        compiler_params=pltpu.CompilerParams(needs_layout_passes=False),
    )
    def agg_kernel(src_hbm, dst_hbm, y_hbm, out_hbm, sidx_v, didx_v, buf_a,
                   buf_b, acc_sh, sem_a, sem_b, sem_i):
        c = lax.axis_index("c")
        s = lax.axis_index("s")
        wid = s * NC + c
        row0 = s * rows_per_tile

        def idxfetch(g, b):
            sl = pl.ds(g * gch, gch)
            pltpu.async_copy(src_hbm.at[wid, sl], sidx_v.at[b], sem_i)
            pltpu.async_copy(dst_hbm.at[wid, sl], didx_v.at[b], sem_i)

        def idxwait(b):
            sl = pl.ds(0, gch)
            pltpu.make_async_copy(src_hbm.at[wid, sl], sidx_v.at[b], sem_i).wait()
            pltpu.make_async_copy(dst_hbm.at[wid, sl], didx_v.at[b], sem_i).wait()

        def gather(b, i, buf, sem):
            pltpu.async_copy(y_hbm.at[sidx_v.at[b, i, 0]], buf, sem)

        def drain(buf, sem):
            pltpu.make_async_copy(y_hbm.at[pl.ds(0, CH)], buf, sem).wait()

        def scatter(b, i, buf):
            pltpu.sync_copy(buf, acc_sh.at[didx_v.at[b, i, 0]], add=True)

        idxfetch(0, 0)

        zero16 = jnp.zeros((LANES,), jnp.float32)

        def zrow(r, carry):
            for k in range(D // LANES):
                buf_a[r, pl.ds(k * LANES, LANES)] = zero16
            return carry

        lax.fori_loop(0, CH, zrow, 0)

        full = rows_per_tile // CH
        rem = rows_per_tile % CH
        for j in range(full):
            pltpu.sync_copy(buf_a, acc_sh.at[pl.ds(row0 + j * CH, CH)])
        if rem:
            pltpu.sync_copy(
                buf_a.at[pl.ds(0, rem)],
                acc_sh.at[pl.ds(row0 + full * CH, rem)],
            )
        idxwait(0)
        plsc.subcore_barrier()

        gather(0, 0, buf_a, sem_a)
        for g in range(G):
            b = g % 2
            if g + 1 < G:
                idxfetch(g + 1, 1 - b)

            def pair(j, carry, b=b):
                i0 = 2 * j
                gather(b, i0 + 1, buf_b, sem_b)
                drain(buf_a, sem_a)
                scatter(b, i0, buf_a)
                gather(b, i0 + 2, buf_a, sem_a)
                drain(buf_b, sem_b)
                scatter(b, i0 + 1, buf_b)
                return carry

            lax.fori_loop(0, (gch - 1) // 2, pair, 0)
            drain(buf_a, sem_a)
            scatter(b, gch - 1, buf_a)
            if g + 1 < G:
                idxwait(1 - b)
                gather(1 - b, 0, buf_a, sem_a)
        plsc.subcore_barrier()

        pltpu.sync_copy(
            acc_sh.at[pl.ds(row0, rows_per_tile)],
            out_hbm.at[c, pl.ds(row0, rows_per_tile)],
        )

    return agg_kernel


def _tc_transform(x, W, histT):
    N, _ = x.shape
    Dout = W.shape[1]

    def body(x_ref, w_ref, h_ref, y_ref, dis_ref):
        deg = jnp.sum(h_ref[...], axis=1, keepdims=True) + 1.0
        dis = lax.rsqrt(deg)
        xw = jnp.dot(x_ref[...], w_ref[...], preferred_element_type=jnp.float32)
        y_ref[...] = xw * dis
        dis_ref[...] = dis

    return pl.pallas_call(
        body,
        out_shape=(
            jax.ShapeDtypeStruct((N, Dout), jnp.float32),
            jax.ShapeDtypeStruct((N, 1), jnp.float32),
        ),
    )(x, W, histT)


def _tc_combine(agg, y, dis):
    N, D = y.shape

    def body(a_ref, y_ref, d_ref, o_ref):
        a = (a_ref[0] + a_ref[1])[:N]
        o_ref[...] = d_ref[...] * (a + y_ref[...])

    return pl.pallas_call(
        body,
        out_shape=jax.ShapeDtypeStruct((N, D), jnp.float32),
    )(agg, y, dis)


def kernel(x, edge_index, W):
    N, _ = x.shape
    Dout = W.shape[1]
    E = edge_index.shape[1]
    assert E % NW == 0 and (E // NW) % LANES == 0
    assert N % NS == 0 and N % LANES == 0 and Dout % LANES == 0

    ei = edge_index.astype(jnp.int32)
    srcs = ei[0]
    dsts = ei[1]

    hist = _deg_kernel(E, N)(dsts)
    histT = hist.reshape(NW, N).T
    y, dis = _tc_transform(x, W, histT)

    e_per_w = E // NW
    n_chunks = -(-e_per_w // CH)
    n_chunks = -(-n_chunks // G_IDX) * G_IDX
    ppw = n_chunks * CH - e_per_w
    rows_per_tile = ((N + NS - 1) // NS + 7) // 8 * 8
    n_spare = rows_per_tile * NS - N
    pad_src = jnp.zeros((NW, ppw), jnp.int32)
    pad_dst = jnp.broadcast_to(
        N + (jnp.arange(NW, dtype=jnp.int32) % n_spare)[:, None], (NW, ppw)
    )
    src3 = jnp.concatenate([srcs.reshape(NW, e_per_w), pad_src], axis=1)
    dst3 = jnp.concatenate([dsts.reshape(NW, e_per_w), pad_dst], axis=1)
    src3 = src3.reshape(NW, n_chunks, 1, CH)
    dst3 = dst3.reshape(NW, n_chunks, 1, CH)
    agg = _agg_kernel(N, Dout, NW * n_chunks * CH)(src3, dst3, y)
    return _tc_combine(agg, y, dis)


# speedup vs baseline: 4.2742x; 1.3007x over previous
import functools

import jax
import jax.numpy as jnp
from jax import lax
from jax.experimental import pallas as pl
from jax.experimental.pallas import tpu as pltpu
from jax.experimental.pallas import tpu_sc as plsc

NC = 2
NS = 16
NW = NC * NS
LANES = 16
CH = 80
G_IDX = 5


def _deg_kernel(E, N):
    e_per_w = E // NW
    mesh = plsc.VectorSubcoreMesh(core_axis_name="c", subcore_axis_name="s")

    @functools.partial(
        pl.kernel,
        out_type=jax.ShapeDtypeStruct((NW, 1, N), jnp.float32),
        mesh=mesh,
        scratch_types=[
            pltpu.VMEM((e_per_w,), jnp.int32),
            pltpu.VMEM((N,), jnp.float32),
        ],
        compiler_params=pltpu.CompilerParams(needs_layout_passes=False),
    )
    def deg_kernel(dst_hbm, hist_hbm, dst_v, hist_v):
        c = lax.axis_index("c")
        s = lax.axis_index("s")
        wid = s * NC + c

        zero16 = jnp.zeros((LANES,), jnp.float32)

        def zbody(i, carry):
            hist_v[pl.ds(i * LANES, LANES)] = zero16
            return carry

        lax.fori_loop(0, N // LANES, zbody, 0)

        pltpu.sync_copy(dst_hbm.at[pl.ds(wid * e_per_w, e_per_w)], dst_v)

        ones16 = jnp.ones((LANES,), jnp.float32)

        def body(i, carry):
            idx = dst_v[pl.ds(i * LANES, LANES)]
            plsc.addupdate_scatter(hist_v, [idx], ones16)
            return carry

        lax.fori_loop(0, e_per_w // LANES, body, 0)

        pltpu.sync_copy(hist_v, hist_hbm.at[wid, 0])

    return deg_kernel


def _agg_kernel(N, D, E):
    e_per_w = E // NW
    n_chunks = e_per_w // CH
    rows_per_tile = ((N + NS - 1) // NS + 7) // 8 * 8
    n_pad = rows_per_tile * NS
    assert n_pad >= N + 1
    mesh = plsc.VectorSubcoreMesh(core_axis_name="c", subcore_axis_name="s")

    G = G_IDX
    gch = n_chunks // G
    assert n_chunks % G == 0 and gch % 2 == 1

    @functools.partial(
        pl.kernel,
        out_type=jax.ShapeDtypeStruct((NC, n_pad, D), jnp.float32),
        mesh=mesh,
        scratch_types=[
            pltpu.VMEM((2, gch, 1, CH), jnp.int32),
            pltpu.VMEM((2, gch, 1, CH), jnp.int32),
            pltpu.VMEM((CH, D), jnp.float32),
            pltpu.VMEM((CH, D), jnp.float32),
            pltpu.VMEM_SHARED((n_pad, D), jnp.float32),
            pltpu.SemaphoreType.DMA,
            pltpu.SemaphoreType.DMA,
            pltpu.SemaphoreType.DMA,
        ],
        compiler_params=pltpu.CompilerParams(needs_layout_passes=False),
    )
    def agg_kernel(src_hbm, dst_hbm, y_hbm, out_hbm, sidx_v, didx_v, buf_a,
                   buf_b, acc_sh, sem_a, sem_b, sem_i):
        c = lax.axis_index("c")
        s = lax.axis_index("s")
        wid = s * NC + c
        row0 = s * rows_per_tile

        def idxfetch(g, b):
            sl = pl.ds(g * gch, gch)
            pltpu.async_copy(src_hbm.at[wid, sl], sidx_v.at[b], sem_i)
            pltpu.async_copy(dst_hbm.at[wid, sl], didx_v.at[b], sem_i)

        def idxwait(b):
            sl = pl.ds(0, gch)
            pltpu.make_async_copy(src_hbm.at[wid, sl], sidx_v.at[b], sem_i).wait()
            pltpu.make_async_copy(dst_hbm.at[wid, sl], didx_v.at[b], sem_i).wait()

        def gather(b, i, buf, sem):
            pltpu.async_copy(y_hbm.at[sidx_v.at[b, i, 0]], buf, sem)

        def drain(buf, sem):
            pltpu.make_async_copy(y_hbm.at[pl.ds(0, CH)], buf, sem).wait()

        def scatter(b, i, buf):
            pltpu.sync_copy(buf, acc_sh.at[didx_v.at[b, i, 0]], add=True)

        idxfetch(0, 0)

        zero16 = jnp.zeros((LANES,), jnp.float32)

        def zrow(r, carry):
            for k in range(D // LANES):
                buf_a[r, pl.ds(k * LANES, LANES)] = zero16
            return carry

        lax.fori_loop(0, CH, zrow, 0)

        full = rows_per_tile // CH
        rem = rows_per_tile % CH
        for j in range(full):
            pltpu.sync_copy(buf_a, acc_sh.at[pl.ds(row0 + j * CH, CH)])
        if rem:
            pltpu.sync_copy(
                buf_a.at[pl.ds(0, rem)],
                acc_sh.at[pl.ds(row0 + full * CH, rem)],
            )
        idxwait(0)
        plsc.subcore_barrier()

        cur, nxt = buf_a, buf_b
        scur, snxt = sem_a, sem_b
        gather(0, 0, cur, scur)
        for g in range(G):
            b = g % 2
            if g + 1 < G:
                idxfetch(g + 1, 1 - b)

            def pair(j, carry, b=b, cur=cur, nxt=nxt, scur=scur, snxt=snxt):
                i0 = 2 * j
                gather(b, i0 + 1, nxt, snxt)
                drain(cur, scur)
                scatter(b, i0, cur)
                gather(b, i0 + 2, cur, scur)
                drain(nxt, snxt)
                scatter(b, i0 + 1, nxt)
                return carry

            lax.fori_loop(0, (gch - 1) // 2, pair, 0)
            if g + 1 < G:
                idxwait(1 - b)
                gather(1 - b, 0, nxt, snxt)
            drain(cur, scur)
            scatter(b, gch - 1, cur)
            cur, nxt = nxt, cur
            scur, snxt = snxt, scur
        plsc.subcore_barrier()

        pltpu.sync_copy(
            acc_sh.at[pl.ds(row0, rows_per_tile)],
            out_hbm.at[c, pl.ds(row0, rows_per_tile)],
        )

    return agg_kernel


def _tc_transform(x, W, histT):
    N, _ = x.shape
    Dout = W.shape[1]

    def body(x_ref, w_ref, h_ref, y_ref, dis_ref):
        deg = jnp.sum(h_ref[...], axis=1, keepdims=True) + 1.0
        dis = lax.rsqrt(deg)
        xw = jnp.dot(x_ref[...], w_ref[...], preferred_element_type=jnp.float32)
        y_ref[...] = xw * dis
        dis_ref[...] = dis

    return pl.pallas_call(
        body,
        out_shape=(
            jax.ShapeDtypeStruct((N, Dout), jnp.float32),
            jax.ShapeDtypeStruct((N, 1), jnp.float32),
        ),
    )(x, W, histT)


def _tc_combine(agg, y, dis):
    N, D = y.shape

    def body(a_ref, y_ref, d_ref, o_ref):
        a = (a_ref[0] + a_ref[1])[:N]
        o_ref[...] = d_ref[...] * (a + y_ref[...])

    return pl.pallas_call(
        body,
        out_shape=jax.ShapeDtypeStruct((N, D), jnp.float32),
    )(agg, y, dis)


def kernel(x, edge_index, W):
    N, _ = x.shape
    Dout = W.shape[1]
    E = edge_index.shape[1]
    assert E % NW == 0 and (E // NW) % LANES == 0
    assert N % NS == 0 and N % LANES == 0 and Dout % LANES == 0

    ei = edge_index.astype(jnp.int32)
    srcs = ei[0]
    dsts = ei[1]

    hist = _deg_kernel(E, N)(dsts)
    histT = hist.reshape(NW, N).T
    y, dis = _tc_transform(x, W, histT)

    e_per_w = E // NW
    n_chunks = -(-e_per_w // CH)
    n_chunks = -(-n_chunks // G_IDX) * G_IDX
    ppw = n_chunks * CH - e_per_w
    rows_per_tile = ((N + NS - 1) // NS + 7) // 8 * 8
    n_spare = rows_per_tile * NS - N
    pad_src = jnp.zeros((NW, ppw), jnp.int32)
    pad_dst = jnp.broadcast_to(
        N + (jnp.arange(NW, dtype=jnp.int32) % n_spare)[:, None], (NW, ppw)
    )
    src3 = jnp.concatenate([srcs.reshape(NW, e_per_w), pad_src], axis=1)
    dst3 = jnp.concatenate([dsts.reshape(NW, e_per_w), pad_dst], axis=1)
    src3 = src3.reshape(NW, n_chunks, 1, CH)
    dst3 = dst3.reshape(NW, n_chunks, 1, CH)
    agg = _agg_kernel(N, Dout, NW * n_chunks * CH)(src3, dst3, y)
    return _tc_combine(agg, y, dis)
